# baseline (speedup 1.0000x reference)
import functools

import jax
import jax.numpy as jnp
from jax import lax
from jax.experimental import pallas as pl
from jax.experimental.pallas import tpu as pltpu
from jax.experimental.pallas import tpu_sc as plsc

N = 10000
E = 320000
D = 128
H = 8
F = 10
HID = 10

NS = 16
NC = 2
NPAD = 10240
EC = E // NS
NV = EC // 16
SLICE = NPAD // NS
HPC = H // NC


def _blockdiag_mask():
    row = lax.broadcasted_iota(jnp.int32, (H, H * F), 0)
    col = lax.broadcasted_iota(jnp.int32, (H, H * F), 1)
    return (col // F == row).astype(jnp.float32)


def _tc_pre_body(x_ref, w_ref, as_ref, ad_ref, ht_ref, asrc_ref, adst_ref,
                 m_ref):
    ht = lax.dot_general(w_ref[...], x_ref[...], (((0,), (1,)), ((), ())),
                         preferred_element_type=jnp.float32)
    ht_ref[...] = ht
    blk = _blockdiag_mask()
    ast = jnp.tile(as_ref[...], (1, H)) * blk
    adt = jnp.tile(ad_ref[...], (1, H)) * blk
    asrc = jnp.dot(ast, ht, preferred_element_type=jnp.float32)
    adst = jnp.dot(adt, ht, preferred_element_type=jnp.float32)
    asrc_ref[...] = asrc
    adst_ref[...] = adst
    sm = (jnp.max(asrc, axis=1, keepdims=True)
          + jnp.max(adst, axis=1, keepdims=True))
    m_ref[...] = jnp.broadcast_to(jnp.maximum(sm, 0.2 * sm), (H, 16))


_tc_pre = pl.pallas_call(
    _tc_pre_body,
    out_shape=[
        jax.ShapeDtypeStruct((H * F, N), jnp.float32),
        jax.ShapeDtypeStruct((H, N), jnp.float32),
        jax.ShapeDtypeStruct((H, N), jnp.float32),
        jax.ShapeDtypeStruct((H, 16), jnp.float32),
    ],
)


def _sc_body(asrc_hbm, adst_hbm, m_hbm, edge_hbm, out_hbm,
             edge_c, p_c, asrc_v, adst_v, den_v, s_v, part_v, red_v,
             red_s_v, m_half, sh_part, sh_den, sem, osem):
    c = lax.axis_index("c")
    s = lax.axis_index("s")
    base = s * EC
    cp_edge = pltpu.async_copy(edge_hbm.at[:, pl.ds(base, EC)], edge_c, sem)
    pltpu.sync_copy(m_hbm.at[pl.ds(c * HPC, HPC)], m_half)
    pltpu.sync_copy(asrc_hbm.at[c * HPC], asrc_v)
    pltpu.sync_copy(adst_hbm.at[c * HPC], adst_v)
    cp_edge.wait()
    src_c = edge_c.at[0]
    dst_c = edge_c.at[1]

    zeros16 = jnp.zeros((16,), jnp.float32)

    def reduce_cols(recip, dst):
        @plsc.parallel_loop(0, SLICE // 16, unroll=2)
        def _(j):
            o = j * 16
            acc = part_v[0, pl.ds(o, 16)]
            for r in range(1, NS):
                acc = acc + part_v[r, pl.ds(o, 16)]
            if recip:
                acc = 1.0 / (acc + 1e-16)
            dst[pl.ds(o, 16)] = acc

    for hh in range(HPC):
        hd = c * HPC + hh
        m16 = m_half[hh]

        with jax.named_scope("zero"):
            @plsc.parallel_loop(0, NPAD // 16, unroll=8)
            def _(i):
                den_v[pl.ds(i * 16, 16)] = zeros16
                s_v[pl.ds(i * 16, 16)] = zeros16

        with jax.named_scope("pass_a"):
            @plsc.parallel_loop(0, NV, unroll=8)
            def _(i):
                o = i * 16
                s16 = src_c[pl.ds(o, 16)]
                d16 = dst_c[pl.ds(o, 16)]
                e = (plsc.load_gather(asrc_v, [s16])
                     + plsc.load_gather(adst_v, [d16]))
                e = jnp.maximum(e, 0.2 * e)
                p = jnp.exp(e - m16)
                p_c[pl.ds(o, 16)] = p
                plsc.addupdate_scatter(den_v, [d16], p)

        if hh + 1 < HPC:
            cp_a = pltpu.async_copy(asrc_hbm.at[hd + 1], asrc_v, sem)
            cp_b = pltpu.async_copy(adst_hbm.at[hd + 1], adst_v, sem)

        if hh > 0:
            plsc.subcore_barrier()

        with jax.named_scope("red_den"):
            pltpu.sync_copy(den_v, sh_part.at[s])
            plsc.subcore_barrier()
            pltpu.sync_copy(sh_part.at[:, pl.ds(s * SLICE, SLICE)], part_v)
            reduce_cols(recip=True, dst=red_v)
            pltpu.sync_copy(red_v, sh_den.at[pl.ds(s * SLICE, SLICE)])
            plsc.subcore_barrier()
            pltpu.sync_copy(sh_den, den_v)

        with jax.named_scope("pass_b"):
            @plsc.parallel_loop(0, NV, unroll=8)
            def _(i):
                o = i * 16
                d16 = dst_c[pl.ds(o, 16)]
                w = p_c[pl.ds(o, 16)] * plsc.load_gather(den_v, [d16])
                s16 = src_c[pl.ds(o, 16)]
                plsc.addupdate_scatter(s_v, [s16], w)

        with jax.named_scope("red_s"):
            pltpu.sync_copy(s_v, sh_part.at[s])
            plsc.subcore_barrier()
            pltpu.sync_copy(sh_part.at[:, pl.ds(s * SLICE, SLICE)], part_v)
            if hh > 0:
                out_cp.wait()
            reduce_cols(recip=False, dst=red_s_v)
            out_cp = pltpu.async_copy(
                red_s_v, out_hbm.at[hd, pl.ds(s * SLICE, SLICE)], osem)
            if hh + 1 < HPC:
                cp_a.wait()
                cp_b.wait()
    out_cp.wait()


def _make_sc_kernel():
    mesh = plsc.VectorSubcoreMesh(core_axis_name="c", subcore_axis_name="s")

    return pl.kernel(
        _sc_body,
        out_type=jax.ShapeDtypeStruct((H, NPAD), jnp.float32),
        mesh=mesh,
        compiler_params=pltpu.CompilerParams(needs_layout_passes=False,
                                             use_tc_tiling_on_sc=False),
        scratch_types=[
            pltpu.VMEM((2, EC), jnp.int32),
            pltpu.VMEM((EC,), jnp.float32),
            pltpu.VMEM((N,), jnp.float32),
            pltpu.VMEM((N,), jnp.float32),
            pltpu.VMEM((NPAD,), jnp.float32),
            pltpu.VMEM((NPAD,), jnp.float32),
            pltpu.VMEM((NS, SLICE), jnp.float32),
            pltpu.VMEM((SLICE,), jnp.float32),
            pltpu.VMEM((SLICE,), jnp.float32),
            pltpu.VMEM((HPC, 16), jnp.float32),
            pltpu.VMEM_SHARED((NS, NPAD), jnp.float32),
            pltpu.VMEM_SHARED((NPAD,), jnp.float32),
            pltpu.SemaphoreType.DMA,
            pltpu.SemaphoreType.DMA,
        ],
    )


_sc_edges = _make_sc_kernel()


def _tc_post_body(st_ref, ht_ref, w1_ref, b1_ref, w2_ref, b2_ref,
                  w3_ref, b3_ref, mask_ref, out_ref):
    big = lax.dot_general(st_ref[:, :N], ht_ref[...],
                          (((1,), (1,)), ((), ())),
                          preferred_element_type=jnp.float32)
    g = jnp.sum(big * _blockdiag_mask(), axis=0, keepdims=True) * (1.0 / N)
    z = jax.nn.sigmoid(
        jnp.dot(g, w1_ref[...], preferred_element_type=jnp.float32)
        + b1_ref[...])
    z = jax.nn.sigmoid(
        jnp.dot(z, w2_ref[...], preferred_element_type=jnp.float32)
        + b2_ref[...])
    logits = (jnp.dot(z, w3_ref[...], preferred_element_type=jnp.float32)
              + b3_ref[...])
    out_ref[...] = jnp.where(mask_ref[...] == 0, jnp.float32(-1.0),
                             logits.reshape(N))


_tc_post = pl.pallas_call(
    _tc_post_body,
    out_shape=jax.ShapeDtypeStruct((N,), jnp.float32),
)


@jax.jit
def kernel(x, edge_index, mask, W, a_src, a_dst, W1, b1, W2, b2, W3, b3):
    ht, asrc_t, adst_t, m_bc = _tc_pre(x, W, a_src, a_dst)
    s_t = _sc_edges(asrc_t, adst_t, m_bc, edge_index)
    return _tc_post(s_t, ht, W1, b1, W2, b2, W3, b3, mask)


# speedup vs baseline: 1.1021x; 1.1021x over previous
import functools

import jax
import jax.numpy as jnp
from jax import lax
from jax.experimental import pallas as pl
from jax.experimental.pallas import tpu as pltpu
from jax.experimental.pallas import tpu_sc as plsc

N = 10000
E = 320000
D = 128
H = 8
F = 10
HID = 10

NS = 16
NC = 2
NPAD = 10240
EC = E // NS
NV = EC // 16
SLICE = NPAD // NS
HPC = H // NC


def _blockdiag_mask():
    row = lax.broadcasted_iota(jnp.int32, (H, H * F), 0)
    col = lax.broadcasted_iota(jnp.int32, (H, H * F), 1)
    return (col // F == row).astype(jnp.float32)


def _tc_pre_body(x_ref, w_ref, as_ref, ad_ref, e_ref, ht_ref, asrc_ref,
                 adst_ref, m_ref, src_ref, dst_ref):
    edge = e_ref[...]
    src_ref[...] = edge[0]
    dst_ref[...] = edge[1]
    ht = lax.dot_general(w_ref[...], x_ref[...], (((0,), (1,)), ((), ())),
                         preferred_element_type=jnp.float32)
    ht_ref[...] = ht
    blk = _blockdiag_mask()
    ast = jnp.tile(as_ref[...], (1, H)) * blk
    adt = jnp.tile(ad_ref[...], (1, H)) * blk
    asrc = jnp.dot(ast, ht, preferred_element_type=jnp.float32)
    adst = jnp.dot(adt, ht, preferred_element_type=jnp.float32)
    asrc_ref[...] = asrc
    adst_ref[...] = adst
    sm = (jnp.max(asrc, axis=1, keepdims=True)
          + jnp.max(adst, axis=1, keepdims=True))
    m_ref[...] = jnp.broadcast_to(jnp.maximum(sm, 0.2 * sm), (H, 16))


_tc_pre = pl.pallas_call(
    _tc_pre_body,
    out_shape=[
        jax.ShapeDtypeStruct((H * F, N), jnp.float32),
        jax.ShapeDtypeStruct((H, N), jnp.float32),
        jax.ShapeDtypeStruct((H, N), jnp.float32),
        jax.ShapeDtypeStruct((H, 16), jnp.float32),
        jax.ShapeDtypeStruct((E,), jnp.int32),
        jax.ShapeDtypeStruct((E,), jnp.int32),
    ],
)


def _sc_body(asrc_hbm, adst_hbm, m_hbm, src_hbm, dst_hbm, out_hbm,
             src_c, dst_c, p_c, asrc_v, adst_v, den_v, s_v, part_v, red_v,
             red_s_v, m_half, sh_part, sh_den, sem, osem):
    c = lax.axis_index("c")
    s = lax.axis_index("s")
    base = s * EC
    cp_src = pltpu.async_copy(src_hbm.at[pl.ds(base, EC)], src_c, sem)
    cp_dst = pltpu.async_copy(dst_hbm.at[pl.ds(base, EC)], dst_c, sem)
    pltpu.sync_copy(m_hbm.at[pl.ds(c * HPC, HPC)], m_half)
    pltpu.sync_copy(asrc_hbm.at[c * HPC], asrc_v)
    pltpu.sync_copy(adst_hbm.at[c * HPC], adst_v)
    cp_src.wait()
    cp_dst.wait()

    zeros16 = jnp.zeros((16,), jnp.float32)

    def reduce_cols(recip, dst):
        @plsc.parallel_loop(0, SLICE // 16, unroll=2)
        def _(j):
            o = j * 16
            acc = part_v[0, pl.ds(o, 16)]
            for r in range(1, NS):
                acc = acc + part_v[r, pl.ds(o, 16)]
            if recip:
                acc = 1.0 / (acc + 1e-16)
            dst[pl.ds(o, 16)] = acc

    for hh in range(HPC):
        hd = c * HPC + hh
        m16 = m_half[hh]

        with jax.named_scope("zero"):
            @plsc.parallel_loop(0, NPAD // 16, unroll=8)
            def _(i):
                den_v[pl.ds(i * 16, 16)] = zeros16
                s_v[pl.ds(i * 16, 16)] = zeros16

        with jax.named_scope("pass_a"):
            @plsc.parallel_loop(0, NV, unroll=8)
            def _(i):
                o = i * 16
                s16 = src_c[pl.ds(o, 16)]
                d16 = dst_c[pl.ds(o, 16)]
                e = (plsc.load_gather(asrc_v, [s16])
                     + plsc.load_gather(adst_v, [d16]))
                e = jnp.maximum(e, 0.2 * e)
                p = jnp.exp(e - m16)
                p_c[pl.ds(o, 16)] = p
                plsc.addupdate_scatter(den_v, [d16], p)

        if hh + 1 < HPC:
            cp_a = pltpu.async_copy(asrc_hbm.at[hd + 1], asrc_v, sem)
            cp_b = pltpu.async_copy(adst_hbm.at[hd + 1], adst_v, sem)

        if hh > 0:
            plsc.subcore_barrier()

        with jax.named_scope("red_den"):
            pltpu.sync_copy(den_v, sh_part.at[s])
            plsc.subcore_barrier()
            pltpu.sync_copy(sh_part.at[:, pl.ds(s * SLICE, SLICE)], part_v)
            reduce_cols(recip=True, dst=red_v)
            pltpu.sync_copy(red_v, sh_den.at[pl.ds(s * SLICE, SLICE)])
            plsc.subcore_barrier()
            pltpu.sync_copy(sh_den, den_v)

        with jax.named_scope("pass_b"):
            @plsc.parallel_loop(0, NV, unroll=8)
            def _(i):
                o = i * 16
                d16 = dst_c[pl.ds(o, 16)]
                w = p_c[pl.ds(o, 16)] * plsc.load_gather(den_v, [d16])
                s16 = src_c[pl.ds(o, 16)]
                plsc.addupdate_scatter(s_v, [s16], w)

        with jax.named_scope("red_s"):
            pltpu.sync_copy(s_v, sh_part.at[s])
            plsc.subcore_barrier()
            pltpu.sync_copy(sh_part.at[:, pl.ds(s * SLICE, SLICE)], part_v)
            if hh > 0:
                out_cp.wait()
            reduce_cols(recip=False, dst=red_s_v)
            out_cp = pltpu.async_copy(
                red_s_v, out_hbm.at[hd, pl.ds(s * SLICE, SLICE)], osem)
            if hh + 1 < HPC:
                cp_a.wait()
                cp_b.wait()
    out_cp.wait()


def _make_sc_kernel():
    mesh = plsc.VectorSubcoreMesh(core_axis_name="c", subcore_axis_name="s")

    return pl.kernel(
        _sc_body,
        out_type=jax.ShapeDtypeStruct((H, NPAD), jnp.float32),
        mesh=mesh,
        compiler_params=pltpu.CompilerParams(needs_layout_passes=False),
        scratch_types=[
            pltpu.VMEM((EC,), jnp.int32),
            pltpu.VMEM((EC,), jnp.int32),
            pltpu.VMEM((EC,), jnp.float32),
            pltpu.VMEM((N,), jnp.float32),
            pltpu.VMEM((N,), jnp.float32),
            pltpu.VMEM((NPAD,), jnp.float32),
            pltpu.VMEM((NPAD,), jnp.float32),
            pltpu.VMEM((NS, SLICE), jnp.float32),
            pltpu.VMEM((SLICE,), jnp.float32),
            pltpu.VMEM((SLICE,), jnp.float32),
            pltpu.VMEM((HPC, 16), jnp.float32),
            pltpu.VMEM_SHARED((NS, NPAD), jnp.float32),
            pltpu.VMEM_SHARED((NPAD,), jnp.float32),
            pltpu.SemaphoreType.DMA,
            pltpu.SemaphoreType.DMA,
        ],
    )


_sc_edges = _make_sc_kernel()


def _tc_post_body(st_ref, ht_ref, w1_ref, b1_ref, w2_ref, b2_ref,
                  w3_ref, b3_ref, mask_ref, out_ref):
    big = lax.dot_general(st_ref[:, :N], ht_ref[...],
                          (((1,), (1,)), ((), ())),
                          preferred_element_type=jnp.float32)
    g = jnp.sum(big * _blockdiag_mask(), axis=0, keepdims=True) * (1.0 / N)
    z = jax.nn.sigmoid(
        jnp.dot(g, w1_ref[...], preferred_element_type=jnp.float32)
        + b1_ref[...])
    z = jax.nn.sigmoid(
        jnp.dot(z, w2_ref[...], preferred_element_type=jnp.float32)
        + b2_ref[...])
    logits = (jnp.dot(z, w3_ref[...], preferred_element_type=jnp.float32)
              + b3_ref[...])
    out_ref[...] = jnp.where(mask_ref[...] == 0, jnp.float32(-1.0),
                             logits.reshape(N))


_tc_post = pl.pallas_call(
    _tc_post_body,
    out_shape=jax.ShapeDtypeStruct((N,), jnp.float32),
)


@jax.jit
def kernel(x, edge_index, mask, W, a_src, a_dst, W1, b1, W2, b2, W3, b3):
    ht, asrc_t, adst_t, m_bc, src, dst = _tc_pre(x, W, a_src, a_dst,
                                                 edge_index)
    s_t = _sc_edges(asrc_t, adst_t, m_bc, src, dst)
    return _tc_post(s_t, ht, W1, b1, W2, b2, W3, b3, mask)


# speedup vs baseline: 1.1858x; 1.0760x over previous
import functools

import jax
import jax.numpy as jnp
from jax import lax
from jax.experimental import pallas as pl
from jax.experimental.pallas import tpu as pltpu
from jax.experimental.pallas import tpu_sc as plsc

N = 10000
E = 320000
D = 128
H = 8
F = 10
HID = 10

NS = 16
NC = 2
NPAD = 10240
EC = E // NS
NV = EC // 16
SLICE = NPAD // NS
HPC = H // NC


def _blockdiag_mask():
    row = lax.broadcasted_iota(jnp.int32, (H, H * F), 0)
    col = lax.broadcasted_iota(jnp.int32, (H, H * F), 1)
    return (col // F == row).astype(jnp.float32)


def _tc_pre_body(x_ref, w_ref, as_ref, ad_ref, e_ref, ht_ref, asrc_ref,
                 adst_ref, m_ref, pk_ref):
    edge = e_ref[...]
    pk_ref[...] = (edge[1] << 16) | edge[0]
    ht = lax.dot_general(w_ref[...], x_ref[...], (((0,), (1,)), ((), ())),
                         preferred_element_type=jnp.float32)
    ht_ref[...] = ht
    blk = _blockdiag_mask()
    ast = jnp.tile(as_ref[...], (1, H)) * blk
    adt = jnp.tile(ad_ref[...], (1, H)) * blk
    asrc = jnp.dot(ast, ht, preferred_element_type=jnp.float32)
    adst = jnp.dot(adt, ht, preferred_element_type=jnp.float32)
    asrc_ref[...] = asrc
    adst_ref[...] = adst
    sm = (jnp.max(asrc, axis=1, keepdims=True)
          + jnp.max(adst, axis=1, keepdims=True))
    m_ref[...] = jnp.broadcast_to(jnp.maximum(sm, 0.2 * sm), (H, 16))


_tc_pre = pl.pallas_call(
    _tc_pre_body,
    out_shape=[
        jax.ShapeDtypeStruct((H * F, N), jnp.float32),
        jax.ShapeDtypeStruct((H, N), jnp.float32),
        jax.ShapeDtypeStruct((H, N), jnp.float32),
        jax.ShapeDtypeStruct((H, 16), jnp.float32),
        jax.ShapeDtypeStruct((E,), jnp.int32),
    ],
)


def _sc_body(asrc_hbm, adst_hbm, m_hbm, pk_hbm, out_hbm,
             pk_c, p_c, asrc_v, adst_v, den_v, s_v, part_v, red_v,
             red_s_v, m_half, sh_part, sh_den, sem, osem):
    c = lax.axis_index("c")
    s = lax.axis_index("s")
    base = s * EC
    cp_pk = pltpu.async_copy(pk_hbm.at[pl.ds(base, EC)], pk_c, sem)
    pltpu.sync_copy(m_hbm.at[pl.ds(c * HPC, HPC)], m_half)
    pltpu.sync_copy(asrc_hbm.at[c * HPC], asrc_v)
    pltpu.sync_copy(adst_hbm.at[c * HPC], adst_v)
    cp_pk.wait()

    zeros16 = jnp.zeros((16,), jnp.float32)

    def reduce_cols(recip, dst):
        @plsc.parallel_loop(0, SLICE // 16, unroll=2)
        def _(j):
            o = j * 16
            acc = part_v[0, pl.ds(o, 16)]
            for r in range(1, NS):
                acc = acc + part_v[r, pl.ds(o, 16)]
            if recip:
                acc = 1.0 / (acc + 1e-16)
            dst[pl.ds(o, 16)] = acc

    for hh in range(HPC):
        hd = c * HPC + hh
        m16 = m_half[hh]

        with jax.named_scope("zero"):
            @plsc.parallel_loop(0, NPAD // 16, unroll=8)
            def _(i):
                den_v[pl.ds(i * 16, 16)] = zeros16
                s_v[pl.ds(i * 16, 16)] = zeros16

        with jax.named_scope("pass_a"):
            @plsc.parallel_loop(0, NV, unroll=8)
            def _(i):
                o = i * 16
                pk = pk_c[pl.ds(o, 16)]
                s16 = pk & 0xFFFF
                d16 = pk >> 16
                e = (plsc.load_gather(asrc_v, [s16])
                     + plsc.load_gather(adst_v, [d16]))
                e = jnp.maximum(e, 0.2 * e)
                p = jnp.exp(e - m16)
                p_c[pl.ds(o, 16)] = p
                plsc.addupdate_scatter(den_v, [d16], p)

        if hh + 1 < HPC:
            cp_a = pltpu.async_copy(asrc_hbm.at[hd + 1], asrc_v, sem)
            cp_b = pltpu.async_copy(adst_hbm.at[hd + 1], adst_v, sem)

        if hh > 0:
            plsc.subcore_barrier()

        with jax.named_scope("red_den"):
            pltpu.sync_copy(den_v, sh_part.at[s])
            plsc.subcore_barrier()
            pltpu.sync_copy(sh_part.at[:, pl.ds(s * SLICE, SLICE)], part_v)
            reduce_cols(recip=True, dst=red_v)
            pltpu.sync_copy(red_v, sh_den.at[pl.ds(s * SLICE, SLICE)])
            plsc.subcore_barrier()
            pltpu.sync_copy(sh_den, den_v)

        with jax.named_scope("pass_b"):
            @plsc.parallel_loop(0, NV, unroll=8)
            def _(i):
                o = i * 16
                pk = pk_c[pl.ds(o, 16)]
                w = p_c[pl.ds(o, 16)] * plsc.load_gather(den_v, [pk >> 16])
                plsc.addupdate_scatter(s_v, [pk & 0xFFFF], w)

        with jax.named_scope("red_s"):
            pltpu.sync_copy(s_v, sh_part.at[s])
            plsc.subcore_barrier()
            pltpu.sync_copy(sh_part.at[:, pl.ds(s * SLICE, SLICE)], part_v)
            if hh > 0:
                out_cp.wait()
            reduce_cols(recip=False, dst=red_s_v)
            out_cp = pltpu.async_copy(
                red_s_v, out_hbm.at[hd, pl.ds(s * SLICE, SLICE)], osem)
            if hh + 1 < HPC:
                cp_a.wait()
                cp_b.wait()
    out_cp.wait()


def _make_sc_kernel():
    mesh = plsc.VectorSubcoreMesh(core_axis_name="c", subcore_axis_name="s")

    return pl.kernel(
        _sc_body,
        out_type=jax.ShapeDtypeStruct((H, NPAD), jnp.float32),
        mesh=mesh,
        compiler_params=pltpu.CompilerParams(needs_layout_passes=False),
        scratch_types=[
            pltpu.VMEM((EC,), jnp.int32),
            pltpu.VMEM((EC,), jnp.float32),
            pltpu.VMEM((N,), jnp.float32),
            pltpu.VMEM((N,), jnp.float32),
            pltpu.VMEM((NPAD,), jnp.float32),
            pltpu.VMEM((NPAD,), jnp.float32),
            pltpu.VMEM((NS, SLICE), jnp.float32),
            pltpu.VMEM((SLICE,), jnp.float32),
            pltpu.VMEM((SLICE,), jnp.float32),
            pltpu.VMEM((HPC, 16), jnp.float32),
            pltpu.VMEM_SHARED((NS, NPAD), jnp.float32),
            pltpu.VMEM_SHARED((NPAD,), jnp.float32),
            pltpu.SemaphoreType.DMA,
            pltpu.SemaphoreType.DMA,
        ],
    )


_sc_edges = _make_sc_kernel()


def _tc_post_body(st_ref, ht_ref, w1_ref, b1_ref, w2_ref, b2_ref,
                  w3_ref, b3_ref, mask_ref, out_ref):
    big = lax.dot_general(st_ref[:, :N], ht_ref[...],
                          (((1,), (1,)), ((), ())),
                          preferred_element_type=jnp.float32)
    g = jnp.sum(big * _blockdiag_mask(), axis=0, keepdims=True) * (1.0 / N)
    z = jax.nn.sigmoid(
        jnp.dot(g, w1_ref[...], preferred_element_type=jnp.float32)
        + b1_ref[...])
    z = jax.nn.sigmoid(
        jnp.dot(z, w2_ref[...], preferred_element_type=jnp.float32)
        + b2_ref[...])
    logits = (jnp.dot(z, w3_ref[...], preferred_element_type=jnp.float32)
              + b3_ref[...])
    out_ref[...] = jnp.where(mask_ref[...] == 0, jnp.float32(-1.0),
                             logits.reshape(N))


_tc_post = pl.pallas_call(
    _tc_post_body,
    out_shape=jax.ShapeDtypeStruct((N,), jnp.float32),
)


@jax.jit
def kernel(x, edge_index, mask, W, a_src, a_dst, W1, b1, W2, b2, W3, b3):
    ht, asrc_t, adst_t, m_bc, pk = _tc_pre(x, W, a_src, a_dst, edge_index)
    s_t = _sc_edges(asrc_t, adst_t, m_bc, pk)
    return _tc_post(s_t, ht, W1, b1, W2, b2, W3, b3, mask)


# speedup vs baseline: 1.1943x; 1.0071x over previous
import functools

import jax
import jax.numpy as jnp
from jax import lax
from jax.experimental import pallas as pl
from jax.experimental.pallas import tpu as pltpu
from jax.experimental.pallas import tpu_sc as plsc

N = 10000
E = 320000
D = 128
H = 8
F = 10
HID = 10

NS = 16
NC = 2
NPAD = 10240
EC = E // NS
NV = EC // 16
SLICE = NPAD // NS
HPC = H // NC


def _blockdiag_mask():
    row = lax.broadcasted_iota(jnp.int32, (H, H * F), 0)
    col = lax.broadcasted_iota(jnp.int32, (H, H * F), 1)
    return (col // F == row).astype(jnp.float32)


def _tc_pre_body(x_ref, w_ref, as_ref, ad_ref, e_ref, ht_ref, asrc_ref,
                 adst_ref, m_ref, pk_ref):
    edge = e_ref[...]
    pk_ref[...] = (edge[1] << 16) | edge[0]
    ht = lax.dot_general(w_ref[...], x_ref[...], (((0,), (1,)), ((), ())),
                         preferred_element_type=jnp.float32)
    ht_ref[...] = ht
    blk = _blockdiag_mask()
    ast = jnp.tile(as_ref[...], (1, H)) * blk
    adt = jnp.tile(ad_ref[...], (1, H)) * blk
    asrc = jnp.dot(ast, ht, preferred_element_type=jnp.float32)
    adst = jnp.dot(adt, ht, preferred_element_type=jnp.float32)
    asrc_ref[...] = asrc
    adst_ref[...] = adst
    sm = (jnp.max(asrc, axis=1, keepdims=True)
          + jnp.max(adst, axis=1, keepdims=True))
    m_ref[...] = jnp.broadcast_to(jnp.maximum(sm, 0.2 * sm), (H, 16))


_tc_pre = pl.pallas_call(
    _tc_pre_body,
    out_shape=[
        jax.ShapeDtypeStruct((H * F, N), jnp.float32),
        jax.ShapeDtypeStruct((H, N), jnp.float32),
        jax.ShapeDtypeStruct((H, N), jnp.float32),
        jax.ShapeDtypeStruct((H, 16), jnp.float32),
        jax.ShapeDtypeStruct((E,), jnp.int32),
    ],
)


def _sc_body(asrc_hbm, adst_hbm, m_hbm, pk_hbm, out_hbm,
             pk_c, p_c, asrc_v, adst_v, den_v, s_v, part_v, red_v,
             red_s_v, m_half, sh_part, sh_den, sem, osem):
    c = lax.axis_index("c")
    s = lax.axis_index("s")
    base = s * EC
    cp_pk = pltpu.async_copy(pk_hbm.at[pl.ds(base, EC)], pk_c, sem)
    pltpu.sync_copy(m_hbm.at[pl.ds(c * HPC, HPC)], m_half)
    pltpu.sync_copy(asrc_hbm.at[c * HPC], asrc_v)
    pltpu.sync_copy(adst_hbm.at[c * HPC], adst_v)

    zeros16 = jnp.zeros((16,), jnp.float32)

    def reduce_cols(recip, dst):
        @plsc.parallel_loop(0, SLICE // 16, unroll=2)
        def _(j):
            o = j * 16
            acc = part_v[0, pl.ds(o, 16)]
            for r in range(1, NS):
                acc = acc + part_v[r, pl.ds(o, 16)]
            if recip:
                acc = 1.0 / (acc + 1e-16)
            dst[pl.ds(o, 16)] = acc

    for hh in range(HPC):
        hd = c * HPC + hh
        m16 = m_half[hh]

        with jax.named_scope("zero"):
            @plsc.parallel_loop(0, NPAD // 16, unroll=8)
            def _(i):
                den_v[pl.ds(i * 16, 16)] = zeros16
                s_v[pl.ds(i * 16, 16)] = zeros16

        if hh == 0:
            cp_pk.wait()

        with jax.named_scope("pass_a"):
            @plsc.parallel_loop(0, NV, unroll=10)
            def _(i):
                o = i * 16
                pk = pk_c[pl.ds(o, 16)]
                s16 = pk & 0xFFFF
                d16 = pk >> 16
                e = (plsc.load_gather(asrc_v, [s16])
                     + plsc.load_gather(adst_v, [d16]))
                e = jnp.maximum(e, 0.2 * e)
                p = jnp.exp(e - m16)
                p_c[pl.ds(o, 16)] = p
                plsc.addupdate_scatter(den_v, [d16], p)

        if hh + 1 < HPC:
            cp_a = pltpu.async_copy(asrc_hbm.at[hd + 1], asrc_v, sem)
            cp_b = pltpu.async_copy(adst_hbm.at[hd + 1], adst_v, sem)

        if hh > 0:
            plsc.subcore_barrier()

        with jax.named_scope("red_den"):
            pltpu.sync_copy(den_v, sh_part.at[s])
            plsc.subcore_barrier()
            pltpu.sync_copy(sh_part.at[:, pl.ds(s * SLICE, SLICE)], part_v)
            reduce_cols(recip=True, dst=red_v)
            pltpu.sync_copy(red_v, sh_den.at[pl.ds(s * SLICE, SLICE)])
            plsc.subcore_barrier()
            pltpu.sync_copy(sh_den, den_v)

        with jax.named_scope("pass_b"):
            @plsc.parallel_loop(0, NV, unroll=10)
            def _(i):
                o = i * 16
                pk = pk_c[pl.ds(o, 16)]
                w = p_c[pl.ds(o, 16)] * plsc.load_gather(den_v, [pk >> 16])
                plsc.addupdate_scatter(s_v, [pk & 0xFFFF], w)

        with jax.named_scope("red_s"):
            pltpu.sync_copy(s_v, sh_part.at[s])
            plsc.subcore_barrier()
            pltpu.sync_copy(sh_part.at[:, pl.ds(s * SLICE, SLICE)], part_v)
            if hh > 0:
                out_cp.wait()
            reduce_cols(recip=False, dst=red_s_v)
            out_cp = pltpu.async_copy(
                red_s_v, out_hbm.at[hd, pl.ds(s * SLICE, SLICE)], osem)
            if hh + 1 < HPC:
                cp_a.wait()
                cp_b.wait()
    out_cp.wait()


def _make_sc_kernel():
    mesh = plsc.VectorSubcoreMesh(core_axis_name="c", subcore_axis_name="s")

    return pl.kernel(
        _sc_body,
        out_type=jax.ShapeDtypeStruct((H, NPAD), jnp.float32),
        mesh=mesh,
        compiler_params=pltpu.CompilerParams(needs_layout_passes=False),
        scratch_types=[
            pltpu.VMEM((EC,), jnp.int32),
            pltpu.VMEM((EC,), jnp.float32),
            pltpu.VMEM((N,), jnp.float32),
            pltpu.VMEM((N,), jnp.float32),
            pltpu.VMEM((NPAD,), jnp.float32),
            pltpu.VMEM((NPAD,), jnp.float32),
            pltpu.VMEM((NS, SLICE), jnp.float32),
            pltpu.VMEM((SLICE,), jnp.float32),
            pltpu.VMEM((SLICE,), jnp.float32),
            pltpu.VMEM((HPC, 16), jnp.float32),
            pltpu.VMEM_SHARED((NS, NPAD), jnp.float32),
            pltpu.VMEM_SHARED((NPAD,), jnp.float32),
            pltpu.SemaphoreType.DMA,
            pltpu.SemaphoreType.DMA,
        ],
    )


_sc_edges = _make_sc_kernel()


def _tc_post_body(st_ref, ht_ref, w1_ref, b1_ref, w2_ref, b2_ref,
                  w3_ref, b3_ref, mask_ref, out_ref):
    big = lax.dot_general(st_ref[:, :N], ht_ref[...],
                          (((1,), (1,)), ((), ())),
                          preferred_element_type=jnp.float32)
    g = jnp.sum(big * _blockdiag_mask(), axis=0, keepdims=True) * (1.0 / N)
    z = jax.nn.sigmoid(
        jnp.dot(g, w1_ref[...], preferred_element_type=jnp.float32)
        + b1_ref[...])
    z = jax.nn.sigmoid(
        jnp.dot(z, w2_ref[...], preferred_element_type=jnp.float32)
        + b2_ref[...])
    logits = (jnp.dot(z, w3_ref[...], preferred_element_type=jnp.float32)
              + b3_ref[...])
    out_ref[...] = jnp.where(mask_ref[...] == 0, jnp.float32(-1.0),
                             logits.reshape(N))


_tc_post = pl.pallas_call(
    _tc_post_body,
    out_shape=jax.ShapeDtypeStruct((N,), jnp.float32),
)


@jax.jit
def kernel(x, edge_index, mask, W, a_src, a_dst, W1, b1, W2, b2, W3, b3):
    ht, asrc_t, adst_t, m_bc, pk = _tc_pre(x, W, a_src, a_dst, edge_index)
    s_t = _sc_edges(asrc_t, adst_t, m_bc, pk)
    return _tc_post(s_t, ht, W1, b1, W2, b2, W3, b3, mask)


# speedup vs baseline: 1.2190x; 1.0206x over previous
import functools

import jax
import jax.numpy as jnp
from jax import lax
from jax.experimental import pallas as pl
from jax.experimental.pallas import tpu as pltpu
from jax.experimental.pallas import tpu_sc as plsc

N = 10000
E = 320000
D = 128
H = 8
F = 10
HID = 10

NS = 16
NC = 2
NPAD = 10240
EC = E // NS
NV = EC // 16
SLICE = NPAD // NS
HPC = H // NC


def _blockdiag_mask():
    row = lax.broadcasted_iota(jnp.int32, (H, H * F), 0)
    col = lax.broadcasted_iota(jnp.int32, (H, H * F), 1)
    return (col // F == row).astype(jnp.float32)


def _tc_pre_body(x_ref, w_ref, as_ref, ad_ref, e_ref, ht_ref, asrc_ref,
                 adst_ref, m_ref, pk_ref):
    edge = e_ref[...]
    pk_ref[...] = (edge[1] << 16) | edge[0]
    ht = lax.dot_general(w_ref[...], x_ref[...], (((1,), (1,)), ((), ())),
                         preferred_element_type=jnp.float32)
    ht_ref[...] = ht
    blk = _blockdiag_mask()
    ast = jnp.tile(as_ref[...], (1, H)) * blk
    adt = jnp.tile(ad_ref[...], (1, H)) * blk
    asrc = jnp.dot(ast, ht, preferred_element_type=jnp.float32)
    adst = jnp.dot(adt, ht, preferred_element_type=jnp.float32)
    asrc_ref[...] = asrc
    adst_ref[...] = adst
    sm = (jnp.max(asrc, axis=1, keepdims=True)
          + jnp.max(adst, axis=1, keepdims=True))
    m_ref[...] = jnp.broadcast_to(jnp.maximum(sm, 0.2 * sm), (H, 16))


_tc_pre = pl.pallas_call(
    _tc_pre_body,
    out_shape=[
        jax.ShapeDtypeStruct((H * F, N), jnp.float32),
        jax.ShapeDtypeStruct((H, N), jnp.float32),
        jax.ShapeDtypeStruct((H, N), jnp.float32),
        jax.ShapeDtypeStruct((H, 16), jnp.float32),
        jax.ShapeDtypeStruct((E,), jnp.int32),
    ],
)


def _sc_body(asrc_hbm, adst_hbm, m_hbm, pk_hbm, out_hbm,
             pk_c, p_c, asrc_v, adst_v, den_v, s_v, part_v, red_v,
             red_s_v, m_half, sh_part, sh_den, sem, osem):
    c = lax.axis_index("c")
    s = lax.axis_index("s")
    base = s * EC
    cp_pk = pltpu.async_copy(pk_hbm.at[pl.ds(base, EC)], pk_c, sem)
    pltpu.sync_copy(m_hbm.at[pl.ds(c * HPC, HPC)], m_half)
    pltpu.sync_copy(asrc_hbm.at[c * HPC], asrc_v)
    pltpu.sync_copy(adst_hbm.at[c * HPC], adst_v)

    zeros16 = jnp.zeros((16,), jnp.float32)

    def reduce_cols(recip, dst):
        @plsc.parallel_loop(0, SLICE // 16, unroll=2)
        def _(j):
            o = j * 16
            acc = part_v[0, pl.ds(o, 16)]
            for r in range(1, NS):
                acc = acc + part_v[r, pl.ds(o, 16)]
            if recip:
                acc = 1.0 / (acc + 1e-16)
            dst[pl.ds(o, 16)] = acc

    for hh in range(HPC):
        hd = c * HPC + hh
        m16 = m_half[hh]

        with jax.named_scope("zero"):
            @plsc.parallel_loop(0, NPAD // 16, unroll=8)
            def _(i):
                den_v[pl.ds(i * 16, 16)] = zeros16
                s_v[pl.ds(i * 16, 16)] = zeros16

        if hh == 0:
            cp_pk.wait()

        with jax.named_scope("pass_a"):
            @plsc.parallel_loop(0, NV, unroll=10)
            def _(i):
                o = i * 16
                pk = pk_c[pl.ds(o, 16)]
                s16 = pk & 0xFFFF
                d16 = pk >> 16
                e = (plsc.load_gather(asrc_v, [s16])
                     + plsc.load_gather(adst_v, [d16]))
                e = jnp.maximum(e, 0.2 * e)
                p = jnp.exp(e - m16)
                p_c[pl.ds(o, 16)] = p
                plsc.addupdate_scatter(den_v, [d16], p)

        if hh + 1 < HPC:
            cp_a = pltpu.async_copy(asrc_hbm.at[hd + 1], asrc_v, sem)
            cp_b = pltpu.async_copy(adst_hbm.at[hd + 1], adst_v, sem)

        if hh > 0:
            plsc.subcore_barrier()

        with jax.named_scope("red_den"):
            pltpu.sync_copy(den_v, sh_part.at[s])
            plsc.subcore_barrier()
            pltpu.sync_copy(sh_part.at[:, pl.ds(s * SLICE, SLICE)], part_v)
            reduce_cols(recip=True, dst=red_v)
            pltpu.sync_copy(red_v, sh_den.at[pl.ds(s * SLICE, SLICE)])
            plsc.subcore_barrier()
            pltpu.sync_copy(sh_den, den_v)

        with jax.named_scope("pass_b"):
            @plsc.parallel_loop(0, NV, unroll=10)
            def _(i):
                o = i * 16
                pk = pk_c[pl.ds(o, 16)]
                w = p_c[pl.ds(o, 16)] * plsc.load_gather(den_v, [pk >> 16])
                plsc.addupdate_scatter(s_v, [pk & 0xFFFF], w)

        with jax.named_scope("red_s"):
            pltpu.sync_copy(s_v, sh_part.at[s])
            plsc.subcore_barrier()
            pltpu.sync_copy(sh_part.at[:, pl.ds(s * SLICE, SLICE)], part_v)
            if hh > 0:
                out_cp.wait()
            reduce_cols(recip=False, dst=red_s_v)
            out_cp = pltpu.async_copy(
                red_s_v, out_hbm.at[hd, pl.ds(s * SLICE, SLICE)], osem)
            if hh + 1 < HPC:
                cp_a.wait()
                cp_b.wait()
    out_cp.wait()


def _make_sc_kernel():
    mesh = plsc.VectorSubcoreMesh(core_axis_name="c", subcore_axis_name="s")

    return pl.kernel(
        _sc_body,
        out_type=jax.ShapeDtypeStruct((H, NPAD), jnp.float32),
        mesh=mesh,
        compiler_params=pltpu.CompilerParams(needs_layout_passes=False),
        scratch_types=[
            pltpu.VMEM((EC,), jnp.int32),
            pltpu.VMEM((EC,), jnp.float32),
            pltpu.VMEM((N,), jnp.float32),
            pltpu.VMEM((N,), jnp.float32),
            pltpu.VMEM((NPAD,), jnp.float32),
            pltpu.VMEM((NPAD,), jnp.float32),
            pltpu.VMEM((NS, SLICE), jnp.float32),
            pltpu.VMEM((SLICE,), jnp.float32),
            pltpu.VMEM((SLICE,), jnp.float32),
            pltpu.VMEM((HPC, 16), jnp.float32),
            pltpu.VMEM_SHARED((NS, NPAD), jnp.float32),
            pltpu.VMEM_SHARED((NPAD,), jnp.float32),
            pltpu.SemaphoreType.DMA,
            pltpu.SemaphoreType.DMA,
        ],
    )


_sc_edges = _make_sc_kernel()


def _tc_post_body(st_ref, ht_ref, w1_ref, b1_ref, w2_ref, b2_ref,
                  w3_ref, b3_ref, mask_ref, out_ref):
    big = lax.dot_general(st_ref[:, :N], ht_ref[...],
                          (((1,), (1,)), ((), ())),
                          preferred_element_type=jnp.float32)
    g = jnp.sum(big * _blockdiag_mask(), axis=0, keepdims=True) * (1.0 / N)
    z = jax.nn.sigmoid(
        lax.dot_general(g, w1_ref[...], (((1,), (1,)), ((), ())),
                        preferred_element_type=jnp.float32)
        + b1_ref[...])
    z = jax.nn.sigmoid(
        jnp.dot(z, w2_ref[...], preferred_element_type=jnp.float32)
        + b2_ref[...])
    logits = (jnp.dot(z, w3_ref[...], preferred_element_type=jnp.float32)
              + b3_ref[...])
    out_ref[...] = jnp.where(mask_ref[...] == 0, jnp.float32(-1.0),
                             logits.reshape(N))


_tc_post = pl.pallas_call(
    _tc_post_body,
    out_shape=jax.ShapeDtypeStruct((N,), jnp.float32),
)


@jax.jit
def kernel(x, edge_index, mask, W, a_src, a_dst, W1, b1, W2, b2, W3, b3):
    ht, asrc_t, adst_t, m_bc, pk = _tc_pre(x, W.T, a_src, a_dst, edge_index)
    s_t = _sc_edges(asrc_t, adst_t, m_bc, pk)
    return _tc_post(s_t, ht, W1.T, b1, W2, b2, W3, b3, mask)


# speedup vs baseline: 1.2382x; 1.0157x over previous
import functools

import jax
import jax.numpy as jnp
from jax import lax
from jax.experimental import pallas as pl
from jax.experimental.pallas import tpu as pltpu
from jax.experimental.pallas import tpu_sc as plsc

N = 10000
E = 320000
D = 128
H = 8
F = 10
HID = 10

NS = 16
NC = 2
NPAD = 10240
EC = E // NS
NV = EC // 16
SLICE = NPAD // NS
HPC = H // NC


def _blockdiag_mask():
    row = lax.broadcasted_iota(jnp.int32, (H, H * F), 0)
    col = lax.broadcasted_iota(jnp.int32, (H, H * F), 1)
    return (col // F == row).astype(jnp.float32)


def _tc_pre_body(x_ref, w_ref, as_ref, ad_ref, e_ref, ht_ref, asrc_ref,
                 adst_ref, m_ref, pk_ref):
    edge = e_ref[...]
    pk_ref[...] = (edge[1] << 16) | edge[0]
    ht = lax.dot_general(w_ref[...], x_ref[...], (((1,), (1,)), ((), ())),
                         preferred_element_type=jnp.float32)
    ht_ref[...] = ht
    blk = _blockdiag_mask()
    ast = jnp.tile(as_ref[...], (1, H)) * blk
    adt = jnp.tile(ad_ref[...], (1, H)) * blk
    asrc = jnp.dot(ast, ht, preferred_element_type=jnp.float32)
    adst = jnp.dot(adt, ht, preferred_element_type=jnp.float32)
    asrc_ref[...] = asrc
    adst_ref[...] = adst
    sm = (jnp.max(asrc, axis=1, keepdims=True)
          + jnp.max(adst, axis=1, keepdims=True))
    m_ref[...] = jnp.broadcast_to(jnp.maximum(sm, 0.2 * sm), (H, 16))


_tc_pre = pl.pallas_call(
    _tc_pre_body,
    out_shape=[
        jax.ShapeDtypeStruct((H * F, N), jnp.float32),
        jax.ShapeDtypeStruct((H, N), jnp.float32),
        jax.ShapeDtypeStruct((H, N), jnp.float32),
        jax.ShapeDtypeStruct((H, 16), jnp.float32),
        jax.ShapeDtypeStruct((E,), jnp.int32),
    ],
)


def _sc_body(asrc_hbm, adst_hbm, m_hbm, pk_hbm, out_hbm,
             pk_c, p_c, asrc_v, adst_v, den_v, s_v, part_v, red_v,
             red_s_v, m_half, sh_part, sh_den, sem, osem, zsem):
    c = lax.axis_index("c")
    s = lax.axis_index("s")
    base = s * EC
    cp_pk = pltpu.async_copy(pk_hbm.at[pl.ds(base, EC)], pk_c, sem)
    pltpu.sync_copy(m_hbm.at[pl.ds(c * HPC, HPC)], m_half)
    pltpu.sync_copy(asrc_hbm.at[c * HPC], asrc_v)
    pltpu.sync_copy(adst_hbm.at[c * HPC], adst_v)

    zeros16 = jnp.zeros((16,), jnp.float32)

    def reduce_cols(recip, dst):
        @plsc.parallel_loop(0, SLICE // 16, unroll=2)
        def _(j):
            o = j * 16
            acc = part_v[0, pl.ds(o, 16)]
            for r in range(1, NS):
                acc = acc + part_v[r, pl.ds(o, 16)]
            if recip:
                acc = 1.0 / (acc + 1e-16)
            dst[pl.ds(o, 16)] = acc

    for hh in range(HPC):
        hd = c * HPC + hh
        m16 = m_half[hh]

        with jax.named_scope("zero"):
            @plsc.parallel_loop(0, NPAD // 16, unroll=8)
            def _(i):
                den_v[pl.ds(i * 16, 16)] = zeros16

        if hh == 0:
            cp_pk.wait()

        with jax.named_scope("pass_a"):
            @plsc.parallel_loop(0, NV, unroll=10)
            def _(i):
                o = i * 16
                pk = pk_c[pl.ds(o, 16)]
                s16 = pk & 0xFFFF
                d16 = pk >> 16
                e = (plsc.load_gather(asrc_v, [s16])
                     + plsc.load_gather(adst_v, [d16]))
                e = jnp.maximum(e, 0.2 * e)
                p = jnp.exp(e - m16)
                p_c[pl.ds(o, 16)] = p
                plsc.addupdate_scatter(den_v, [d16], p)

        if hh + 1 < HPC:
            cp_a = pltpu.async_copy(asrc_hbm.at[hd + 1], asrc_v, sem)
            cp_b = pltpu.async_copy(adst_hbm.at[hd + 1], adst_v, sem)

        if hh > 0:
            plsc.subcore_barrier()

        with jax.named_scope("red_den"):
            cp_d = pltpu.async_copy(den_v, sh_part.at[s], zsem)
            @plsc.parallel_loop(0, NPAD // 16, unroll=8)
            def _(i):
                s_v[pl.ds(i * 16, 16)] = zeros16
            cp_d.wait()
            plsc.subcore_barrier()
            pltpu.sync_copy(sh_part.at[:, pl.ds(s * SLICE, SLICE)], part_v)
            reduce_cols(recip=True, dst=red_v)
            pltpu.sync_copy(red_v, sh_den.at[pl.ds(s * SLICE, SLICE)])
            plsc.subcore_barrier()
            pltpu.sync_copy(sh_den, den_v)

        with jax.named_scope("pass_b"):
            @plsc.parallel_loop(0, NV, unroll=10)
            def _(i):
                o = i * 16
                pk = pk_c[pl.ds(o, 16)]
                w = p_c[pl.ds(o, 16)] * plsc.load_gather(den_v, [pk >> 16])
                plsc.addupdate_scatter(s_v, [pk & 0xFFFF], w)

        with jax.named_scope("red_s"):
            pltpu.sync_copy(s_v, sh_part.at[s])
            plsc.subcore_barrier()
            pltpu.sync_copy(sh_part.at[:, pl.ds(s * SLICE, SLICE)], part_v)
            if hh > 0:
                out_cp.wait()
            reduce_cols(recip=False, dst=red_s_v)
            out_cp = pltpu.async_copy(
                red_s_v, out_hbm.at[hd, pl.ds(s * SLICE, SLICE)], osem)
            if hh + 1 < HPC:
                cp_a.wait()
                cp_b.wait()
    out_cp.wait()


def _make_sc_kernel():
    mesh = plsc.VectorSubcoreMesh(core_axis_name="c", subcore_axis_name="s")

    return pl.kernel(
        _sc_body,
        out_type=jax.ShapeDtypeStruct((H, NPAD), jnp.float32),
        mesh=mesh,
        compiler_params=pltpu.CompilerParams(needs_layout_passes=False),
        scratch_types=[
            pltpu.VMEM((EC,), jnp.int32),
            pltpu.VMEM((EC,), jnp.float32),
            pltpu.VMEM((N,), jnp.float32),
            pltpu.VMEM((N,), jnp.float32),
            pltpu.VMEM((NPAD,), jnp.float32),
            pltpu.VMEM((NPAD,), jnp.float32),
            pltpu.VMEM((NS, SLICE), jnp.float32),
            pltpu.VMEM((SLICE,), jnp.float32),
            pltpu.VMEM((SLICE,), jnp.float32),
            pltpu.VMEM((HPC, 16), jnp.float32),
            pltpu.VMEM_SHARED((NS, NPAD), jnp.float32),
            pltpu.VMEM_SHARED((NPAD,), jnp.float32),
            pltpu.SemaphoreType.DMA,
            pltpu.SemaphoreType.DMA,
            pltpu.SemaphoreType.DMA,
        ],
    )


_sc_edges = _make_sc_kernel()


def _tc_post_body(st_ref, ht_ref, w1_ref, b1_ref, w2_ref, b2_ref,
                  w3_ref, b3_ref, mask_ref, out_ref):
    big = lax.dot_general(st_ref[:, :N], ht_ref[...],
                          (((1,), (1,)), ((), ())),
                          preferred_element_type=jnp.float32)
    g = jnp.sum(big * _blockdiag_mask(), axis=0, keepdims=True) * (1.0 / N)
    z = jax.nn.sigmoid(
        lax.dot_general(g, w1_ref[...], (((1,), (1,)), ((), ())),
                        preferred_element_type=jnp.float32)
        + b1_ref[...])
    z = jax.nn.sigmoid(
        jnp.dot(z, w2_ref[...], preferred_element_type=jnp.float32)
        + b2_ref[...])
    logits = (jnp.dot(z, w3_ref[...], preferred_element_type=jnp.float32)
              + b3_ref[...])
    out_ref[...] = jnp.where(mask_ref[...] == 0, jnp.float32(-1.0),
                             logits.reshape(N))


_tc_post = pl.pallas_call(
    _tc_post_body,
    out_shape=jax.ShapeDtypeStruct((N,), jnp.float32),
)


@jax.jit
def kernel(x, edge_index, mask, W, a_src, a_dst, W1, b1, W2, b2, W3, b3):
    ht, asrc_t, adst_t, m_bc, pk = _tc_pre(x, W.T, a_src, a_dst, edge_index)
    s_t = _sc_edges(asrc_t, adst_t, m_bc, pk)
    return _tc_post(s_t, ht, W1.T, b1, W2, b2, W3, b3, mask)
